# baseline (speedup 1.0000x reference)
import functools

import jax
import jax.numpy as jnp
from jax import lax
from jax.experimental import pallas as pl
from jax.experimental.pallas import tpu as pltpu
from jax.experimental.pallas import tpu_sc as plsc

T = 2048
C = 768
H = 12
KVH = 4
HD = C // H
E = 8
K = 2
V = 32000
EPS = 1e-06

VOCAB_BLK = 1280


def _rms(x, w, eps=EPS):
    return x * lax.rsqrt(jnp.mean(x * x, axis=-1, keepdims=True) + eps) * w


def _bdot(a, b, dims):
    return lax.dot_general(a.astype(jnp.bfloat16), b.astype(jnp.bfloat16),
                           (dims, ((), ())),
                           preferred_element_type=jnp.float32)


@functools.lru_cache(maxsize=1)
def _make_sc_gather():
    info = plsc.get_sparse_core_info()
    nw = info.num_cores * info.num_subcores
    b_per_w = T // nw

    mesh = plsc.VectorSubcoreMesh(core_axis_name="c", subcore_axis_name="s")

    @functools.partial(
        pl.kernel,
        mesh=mesh,
        out_type=jax.ShapeDtypeStruct((T, C), jnp.float32),
        scratch_types=[
            pltpu.VMEM((b_per_w,), jnp.int32),
            pltpu.VMEM((b_per_w, C), jnp.float32),
            pltpu.SemaphoreType.DMA,
        ],
    )
    def gather_rows(ids_hbm, table_hbm, out_hbm, idx_v, rows_v, sem):
        wid = lax.axis_index("s") * info.num_cores + lax.axis_index("c")
        base = wid * b_per_w
        pltpu.sync_copy(ids_hbm.at[pl.ds(base, b_per_w)], idx_v)
        pltpu.async_copy(table_hbm.at[idx_v], rows_v, sem).wait()
        pltpu.sync_copy(rows_v, out_hbm.at[pl.ds(base, b_per_w)])

    return gather_rows


def _moe_body(hn_ref, h2_ref, wmat_ref, ew_ref, fw_ref, out_ref):
    e = pl.program_id(0)
    hn = hn_ref[...]
    wmat = wmat_ref[...]
    lane = lax.broadcasted_iota(jnp.int32, wmat.shape, 1)
    we = jnp.sum(jnp.where(lane == e, wmat, 0.0), axis=1, keepdims=True)

    contrib = we * _bdot(hn, ew_ref[0], ((1,), (1,)))

    @pl.when(e == 0)
    def _():
        out_ref[...] = contrib

    @pl.when(e > 0)
    def _():
        out_ref[...] += contrib

    @pl.when(e == E - 1)
    def _():
        out_ref[...] = _rms(h2_ref[...] + out_ref[...], fw_ref[...])


def _moe(hn, h2, wmat, ew, fw):
    full2 = lambda e: (0, 0)
    return pl.pallas_call(
        _moe_body,
        grid=(E,),
        in_specs=[
            pl.BlockSpec((T, C), full2),
            pl.BlockSpec((T, C), full2),
            pl.BlockSpec((T, E), full2),
            pl.BlockSpec((1, C, C), lambda e: (e, 0, 0)),
            pl.BlockSpec((1, C), full2),
        ],
        out_specs=pl.BlockSpec((T, C), full2),
        out_shape=jax.ShapeDtypeStruct((T, C), jnp.float32),
        compiler_params=pltpu.CompilerParams(
            vmem_limit_bytes=100 * 1024 * 1024),
    )(hn, h2, wmat, ew, fw)


def _head_body(xf_ref, w_ref, out_ref):
    out_ref[...] = _bdot(xf_ref[...], w_ref[...], ((1,), (1,)))


def _lm_head(xf, w):
    return pl.pallas_call(
        _head_body,
        grid=(V // VOCAB_BLK,),
        in_specs=[
            pl.BlockSpec((T, C), lambda i: (0, 0)),
            pl.BlockSpec((VOCAB_BLK, C), lambda i: (i, 0)),
        ],
        out_specs=pl.BlockSpec((T, VOCAB_BLK), lambda i: (0, i)),
        out_shape=jax.ShapeDtypeStruct((T, V), jnp.float32),
        compiler_params=pltpu.CompilerParams(
            vmem_limit_bytes=60 * 1024 * 1024),
    )(xf, w)


def kernel(input_ids, tok_emb, pos_emb, Wq, Wk, Wv, Wo, attn_norm_w,
           ffn_norm_w, gate_W, expert_W, final_norm_w, out_W):
    ids = input_ids.reshape(T).astype(jnp.int32)
    g = _make_sc_gather()(ids, tok_emb)
    h = g + pos_emb[:T]

    xn = _rms(h, attn_norm_w[0])
    b, t = 1, T
    x3 = xn[None]
    xq = (x3 @ Wq[0].T).reshape(b, t, H, HD)
    xk = (x3 @ Wk[0].T).reshape(b, t, KVH, HD)
    xv = (x3 @ Wv[0].T).reshape(b, t, KVH, HD)
    n_rep = H // KVH
    xk = jnp.repeat(xk, n_rep, axis=2).transpose(0, 2, 1, 3)
    xv = jnp.repeat(xv, n_rep, axis=2).transpose(0, 2, 1, 3)
    xq = xq.transpose(0, 2, 1, 3)
    attn = (xq @ xk.transpose(0, 1, 3, 2)) * (1.0 / HD ** 0.5)
    mask = jnp.tril(jnp.ones((t, t), dtype=jnp.float32))[None, None, :, :]
    attn = jnp.where(mask == 0, -jnp.inf, attn)
    attn = jax.nn.softmax(attn, axis=-1)
    r = (attn @ xv).transpose(0, 2, 1, 3).reshape(b, t, C) @ Wo[0].T
    h2 = h + r[0]
    hn = _rms(h2, ffn_norm_w[0])
    glog = hn @ gate_W[0].T
    weights, selected = lax.top_k(glog, K)
    weights = jax.nn.softmax(weights.astype(jnp.float32), axis=1)
    wmat = jnp.stack(
        [jnp.sum(weights * (selected == i).astype(jnp.float32), axis=1)
         for i in range(E)], axis=1)

    xf = _moe(hn, h2, wmat, expert_W[0], final_norm_w.reshape(1, C))
    logits = _lm_head(xf, out_W)
    return logits.reshape(1, T, V)


# speedup vs baseline: 1.0050x; 1.0050x over previous
import functools

import jax
import jax.numpy as jnp
from jax import lax
from jax.experimental import pallas as pl
from jax.experimental.pallas import tpu as pltpu
from jax.experimental.pallas import tpu_sc as plsc

T = 2048
C = 768
H = 12
KVH = 4
HD = C // H
E = 8
K = 2
V = 32000
EPS = 1e-06

VOCAB_BLK = 1280


def _rms(x, w, eps=EPS):
    return x * lax.rsqrt(jnp.mean(x * x, axis=-1, keepdims=True) + eps) * w


def _bdot(a, b, dims):
    return lax.dot_general(a.astype(jnp.bfloat16), b.astype(jnp.bfloat16),
                           (dims, ((), ())),
                           preferred_element_type=jnp.float32)


@functools.lru_cache(maxsize=1)
def _make_sc_gather():
    info = plsc.get_sparse_core_info()
    nw = info.num_cores * info.num_subcores
    b_per_w = T // nw

    mesh = plsc.VectorSubcoreMesh(core_axis_name="c", subcore_axis_name="s")

    @functools.partial(
        pl.kernel,
        mesh=mesh,
        out_type=jax.ShapeDtypeStruct((T, C), jnp.float32),
        scratch_types=[
            pltpu.VMEM((b_per_w,), jnp.int32),
            pltpu.VMEM((b_per_w, C), jnp.float32),
            pltpu.SemaphoreType.DMA,
        ],
    )
    def gather_rows(ids_hbm, table_hbm, out_hbm, idx_v, rows_v, sem):
        wid = lax.axis_index("s") * info.num_cores + lax.axis_index("c")
        base = wid * b_per_w
        pltpu.sync_copy(ids_hbm.at[pl.ds(base, b_per_w)], idx_v)
        pltpu.async_copy(table_hbm.at[idx_v], rows_v, sem).wait()
        pltpu.sync_copy(rows_v, out_hbm.at[pl.ds(base, b_per_w)])

    return gather_rows


def _moe_body(hn_ref, h2_ref, wmat_ref, ew_ref, fw_ref, out_ref, acc_ref):
    e = pl.program_id(0)
    hn = hn_ref[...]
    wmat = wmat_ref[...]
    lane = lax.broadcasted_iota(jnp.int32, wmat.shape, 1)
    we = jnp.sum(jnp.where(lane == e, wmat, 0.0), axis=1, keepdims=True)

    contrib = we * _bdot(hn, ew_ref[0], ((1,), (1,)))

    @pl.when(e == 0)
    def _():
        acc_ref[...] = contrib

    @pl.when(e > 0)
    def _():
        acc_ref[...] += contrib

    @pl.when(e == E - 1)
    def _():
        out_ref[...] = _rms(h2_ref[...] + acc_ref[...],
                            fw_ref[...]).astype(jnp.bfloat16)


def _moe(hn, h2, wmat, ew, fw):
    full2 = lambda e: (0, 0)
    return pl.pallas_call(
        _moe_body,
        grid=(E,),
        in_specs=[
            pl.BlockSpec((T, C), full2),
            pl.BlockSpec((T, C), full2),
            pl.BlockSpec((T, E), full2),
            pl.BlockSpec((1, C, C), lambda e: (e, 0, 0)),
            pl.BlockSpec((1, C), full2),
        ],
        out_specs=pl.BlockSpec((T, C), full2),
        out_shape=jax.ShapeDtypeStruct((T, C), jnp.bfloat16),
        scratch_shapes=[pltpu.VMEM((T, C), jnp.float32)],
        compiler_params=pltpu.CompilerParams(
            vmem_limit_bytes=100 * 1024 * 1024),
    )(hn, h2, wmat, ew, fw)


def _head_body(xf_ref, w_ref, out_ref):
    out_ref[...] = _bdot(xf_ref[...], w_ref[...], ((1,), (1,)))


def _lm_head(xf, w):
    return pl.pallas_call(
        _head_body,
        grid=(V // VOCAB_BLK,),
        in_specs=[
            pl.BlockSpec((T, C), lambda i: (0, 0)),
            pl.BlockSpec((VOCAB_BLK, C), lambda i: (i, 0)),
        ],
        out_specs=pl.BlockSpec((T, VOCAB_BLK), lambda i: (0, i)),
        out_shape=jax.ShapeDtypeStruct((T, V), jnp.float32),
        compiler_params=pltpu.CompilerParams(
            vmem_limit_bytes=60 * 1024 * 1024),
    )(xf, w)


def kernel(input_ids, tok_emb, pos_emb, Wq, Wk, Wv, Wo, attn_norm_w,
           ffn_norm_w, gate_W, expert_W, final_norm_w, out_W):
    ids = input_ids.reshape(T).astype(jnp.int32)
    g = _make_sc_gather()(ids, tok_emb)
    h = g + pos_emb[:T]

    xn = _rms(h, attn_norm_w[0])
    b, t = 1, T
    x3 = xn[None]
    xq = (x3 @ Wq[0].T).reshape(b, t, H, HD)
    xk = (x3 @ Wk[0].T).reshape(b, t, KVH, HD)
    xv = (x3 @ Wv[0].T).reshape(b, t, KVH, HD)
    n_rep = H // KVH
    xk = jnp.repeat(xk, n_rep, axis=2).transpose(0, 2, 1, 3)
    xv = jnp.repeat(xv, n_rep, axis=2).transpose(0, 2, 1, 3)
    xq = xq.transpose(0, 2, 1, 3)
    attn = (xq @ xk.transpose(0, 1, 3, 2)) * (1.0 / HD ** 0.5)
    mask = jnp.tril(jnp.ones((t, t), dtype=jnp.float32))[None, None, :, :]
    attn = jnp.where(mask == 0, -jnp.inf, attn)
    attn = jax.nn.softmax(attn, axis=-1)
    r = (attn @ xv).transpose(0, 2, 1, 3).reshape(b, t, C) @ Wo[0].T
    h2 = h + r[0]
    hn = _rms(h2, ffn_norm_w[0])
    glog = hn @ gate_W[0].T
    weights, selected = lax.top_k(glog, K)
    weights = jax.nn.softmax(weights.astype(jnp.float32), axis=1)
    wmat = jnp.stack(
        [jnp.sum(weights * (selected == i).astype(jnp.float32), axis=1)
         for i in range(E)], axis=1)

    xf = _moe(hn, h2, wmat, expert_W[0], final_norm_w.reshape(1, C))
    logits = _lm_head(xf, out_W)
    return logits.reshape(1, T, V)
